# Initial kernel scaffold; baseline (speedup 1.0000x reference)
import jax
import jax.numpy as jnp
from jax.experimental import pallas as pl


def kernel(x, W1, W2, train_set_left, train_set_right):
    raise NotImplementedError("write your pallas kernel here")


# speedup vs baseline: 1.5792x; 1.5792x over previous
import functools

import jax
import jax.numpy as jnp
from jax import lax
from jax.experimental import pallas as pl
from jax.experimental.pallas import tpu as pltpu
from jax.experimental.pallas import tpu_sc as plsc

D = 256
M = 50000
NC = 2
NS = 16
NW = NC * NS
BPW = 1568
M_PAD = NW * BPW
CH = 112
NCHUNK = BPW // CH
BM = 1024
GRID = M_PAD // BM


def _sc_gather(x, left_pad, right_pad):
    mesh = plsc.VectorSubcoreMesh(core_axis_name="c", subcore_axis_name="s")

    @functools.partial(
        pl.kernel,
        out_type=[jax.ShapeDtypeStruct((M_PAD, D), jnp.float32),
                  jax.ShapeDtypeStruct((M_PAD, D), jnp.float32)],
        mesh=mesh,
        scratch_types=[
            pltpu.VMEM((BPW,), jnp.int32),
            pltpu.VMEM((BPW,), jnp.int32),
            pltpu.VMEM((CH, D), jnp.float32),
            pltpu.VMEM((CH, D), jnp.float32),
            pltpu.SemaphoreType.DMA,
            pltpu.SemaphoreType.DMA,
        ],
    )
    def k(x_hbm, l_hbm, r_hbm, out_l, out_r, idx_l, idx_r, buf_l, buf_r,
          sem_l, sem_r):
        wid = lax.axis_index("s") * NC + lax.axis_index("c")
        base = wid * BPW
        pltpu.sync_copy(l_hbm.at[pl.ds(base, BPW)], idx_l)
        pltpu.sync_copy(r_hbm.at[pl.ds(base, BPW)], idx_r)

        def body(c, carry):
            off = c * CH
            cl = pltpu.async_copy(x_hbm.at[idx_l.at[pl.ds(off, CH)]], buf_l,
                                  sem_l)
            cr = pltpu.async_copy(x_hbm.at[idx_r.at[pl.ds(off, CH)]], buf_r,
                                  sem_r)
            cl.wait()
            pltpu.sync_copy(buf_l, out_l.at[pl.ds(base + off, CH)])
            cr.wait()
            pltpu.sync_copy(buf_r, out_r.at[pl.ds(base + off, CH)])
            return carry

        lax.fori_loop(0, NCHUNK, body, 0)

    return k(x, left_pad, right_pad)


def _loss_body(xl_ref, xr_ref, w1_ref, w2_ref, out_ref, g_ref, acc_ref):
    i = pl.program_id(0)

    @pl.when(i == 0)
    def _init():
        w1 = w1_ref[...]
        w2 = w2_ref[...]
        dn = (((1,), (1,)), ((), ()))
        g_ref[:, 0:D] = lax.dot_general(w1, w2, dn,
                                        preferred_element_type=jnp.float32)
        g_ref[:, D:2 * D] = lax.dot_general(w1, w1, dn,
                                            preferred_element_type=jnp.float32)
        g_ref[:, 2 * D:3 * D] = lax.dot_general(
            w2, w2, dn, preferred_element_type=jnp.float32)
        acc_ref[0] = 0.0

    xl = xl_ref[...]
    xr = xr_ref[...]
    a = jnp.dot(xl, g_ref[:, 0:2 * D], preferred_element_type=jnp.float32)
    b = jnp.dot(xr, g_ref[:, 2 * D:3 * D], preferred_element_type=jnp.float32)
    dot = jnp.sum(a[:, 0:D] * xr, axis=1, keepdims=True)
    n1 = jnp.sum(a[:, D:2 * D] * xl, axis=1, keepdims=True)
    n2 = jnp.sum(b * xr, axis=1, keepdims=True)
    denom = jnp.sqrt(jnp.maximum(n1, 0.0) * jnp.maximum(n2, 0.0))
    cos = dot / jnp.maximum(denom, 1e-8)
    r = cos - 1.0
    row = i * BM + lax.broadcasted_iota(jnp.int32, (BM, 1), 0)
    sq = jnp.where(row < M, r * r, 0.0)
    acc_ref[0] += jnp.sum(sq)

    @pl.when(i == GRID - 1)
    def _fin():
        out_ref[0] = acc_ref[0] * (1.0 / M)


def _tc_loss(xl, xr, W1, W2):
    return pl.pallas_call(
        _loss_body,
        grid=(GRID,),
        in_specs=[
            pl.BlockSpec((BM, D), lambda i: (i, 0)),
            pl.BlockSpec((BM, D), lambda i: (i, 0)),
            pl.BlockSpec((D, D), lambda i: (0, 0)),
            pl.BlockSpec((D, D), lambda i: (0, 0)),
        ],
        out_specs=pl.BlockSpec(memory_space=pltpu.SMEM),
        out_shape=jax.ShapeDtypeStruct((1,), jnp.float32),
        scratch_shapes=[pltpu.VMEM((D, 3 * D), jnp.float32),
                        pltpu.SMEM((1,), jnp.float32)],
    )(xl, xr, W1, W2)


def kernel(x, W1, W2, train_set_left, train_set_right):
    left = train_set_left.astype(jnp.int32)
    right = train_set_right.astype(jnp.int32)
    pad = M_PAD - M
    left = jnp.concatenate([left, jnp.zeros((pad,), jnp.int32)])
    right = jnp.concatenate([right, jnp.zeros((pad,), jnp.int32)])
    xl, xr = _sc_gather(x, left, right)
    loss = _tc_loss(xl, xr, W1, W2)
    return loss[0]


# speedup vs baseline: 1.7298x; 1.0954x over previous
import functools

import jax
import jax.numpy as jnp
from jax import lax
from jax.experimental import pallas as pl
from jax.experimental.pallas import tpu as pltpu
from jax.experimental.pallas import tpu_sc as plsc

D = 256
M = 50000
NC = 2
NS = 16
NW = NC * NS
M_PAD = 50176
CHUNKS = 4
CM = M_PAD // CHUNKS
RPW = CM // NW
CH = 56
NCH = RPW // CH
BM = 896
GRID = CM // BM


def _make_sc_gather(qoff):
    mesh = plsc.VectorSubcoreMesh(core_axis_name="c", subcore_axis_name="s")

    @functools.partial(
        pl.kernel,
        out_type=[jax.ShapeDtypeStruct((CM, D), jnp.float32),
                  jax.ShapeDtypeStruct((CM, D), jnp.float32)],
        mesh=mesh,
        scratch_types=[
            pltpu.VMEM((RPW,), jnp.int32),
            pltpu.VMEM((RPW,), jnp.int32),
            pltpu.VMEM((2, CH, D), jnp.float32),
            pltpu.VMEM((2, CH, D), jnp.float32),
            pltpu.SemaphoreType.DMA,
            pltpu.SemaphoreType.DMA,
            pltpu.SemaphoreType.DMA,
            pltpu.SemaphoreType.DMA,
        ],
    )
    def k(x_hbm, l_hbm, r_hbm, out_l, out_r, idx_l, idx_r, buf_l, buf_r,
          sl0, sl1, sr0, sr1):
        wid = lax.axis_index("s") * NC + lax.axis_index("c")
        base = wid * RPW
        pltpu.sync_copy(l_hbm.at[pl.ds(qoff + base, RPW)], idx_l)
        pltpu.sync_copy(r_hbm.at[pl.ds(qoff + base, RPW)], idx_r)
        sems_l = (sl0, sl1)
        sems_r = (sr0, sr1)

        def start(c):
            p = c % 2
            cl = pltpu.async_copy(x_hbm.at[idx_l.at[pl.ds(c * CH, CH)]],
                                  buf_l.at[p], sems_l[p])
            cr = pltpu.async_copy(x_hbm.at[idx_r.at[pl.ds(c * CH, CH)]],
                                  buf_r.at[p], sems_r[p])
            return cl, cr

        pend = start(0)
        for c in range(NCH):
            cl, cr = pend
            if c + 1 < NCH:
                pend = start(c + 1)
            cl.wait()
            pltpu.sync_copy(buf_l.at[c % 2],
                            out_l.at[pl.ds(base + c * CH, CH)])
            cr.wait()
            pltpu.sync_copy(buf_r.at[c % 2],
                            out_r.at[pl.ds(base + c * CH, CH)])

    return k


def _gram_body(w1_ref, w2_ref, g_ref):
    w1 = w1_ref[...]
    w2 = w2_ref[...]
    dn = (((1,), (1,)), ((), ()))
    g_ref[:, 0:D] = lax.dot_general(w1, w2, dn,
                                    preferred_element_type=jnp.float32)
    g_ref[:, D:2 * D] = lax.dot_general(w1, w1, dn,
                                        preferred_element_type=jnp.float32)
    g_ref[:, 2 * D:3 * D] = lax.dot_general(w2, w2, dn,
                                            preferred_element_type=jnp.float32)


def _gram(W1, W2):
    return pl.pallas_call(
        _gram_body,
        out_shape=jax.ShapeDtypeStruct((D, 3 * D), jnp.float32),
    )(W1, W2)


def _partial_body(off, masked, xl_ref, xr_ref, g_ref, out_ref, acc_ref):
    i = pl.program_id(0)

    @pl.when(i == 0)
    def _init():
        acc_ref[0] = 0.0

    xl = xl_ref[...]
    xr = xr_ref[...]
    a = jnp.dot(xl, g_ref[:, 0:2 * D], preferred_element_type=jnp.float32)
    b = jnp.dot(xr, g_ref[:, 2 * D:3 * D], preferred_element_type=jnp.float32)
    dot = jnp.sum(a[:, 0:D] * xr, axis=1, keepdims=True)
    n1 = jnp.sum(a[:, D:2 * D] * xl, axis=1, keepdims=True)
    n2 = jnp.sum(b * xr, axis=1, keepdims=True)
    denom = jnp.sqrt(jnp.maximum(n1, 0.0) * jnp.maximum(n2, 0.0))
    cos = dot / jnp.maximum(denom, 1e-8)
    r = cos - 1.0
    if masked:
        row = off + i * BM + lax.broadcasted_iota(jnp.int32, (BM, 1), 0)
        sq = jnp.where(row < M, r * r, 0.0)
    else:
        sq = r * r
    acc_ref[0] += jnp.sum(sq)

    @pl.when(i == GRID - 1)
    def _fin():
        out_ref[0] = acc_ref[0]


def _tc_partial(xl, xr, g, off, masked):
    return pl.pallas_call(
        functools.partial(_partial_body, off, masked),
        grid=(GRID,),
        in_specs=[
            pl.BlockSpec((BM, D), lambda i: (i, 0)),
            pl.BlockSpec((BM, D), lambda i: (i, 0)),
            pl.BlockSpec((D, 3 * D), lambda i: (0, 0)),
        ],
        out_specs=pl.BlockSpec(memory_space=pltpu.SMEM),
        out_shape=jax.ShapeDtypeStruct((1,), jnp.float32),
        scratch_shapes=[pltpu.SMEM((1,), jnp.float32)],
    )(xl, xr, g)


def kernel(x, W1, W2, train_set_left, train_set_right):
    left = train_set_left.astype(jnp.int32)
    right = train_set_right.astype(jnp.int32)
    pad = M_PAD - M
    left = jnp.concatenate([left, jnp.zeros((pad,), jnp.int32)])
    right = jnp.concatenate([right, jnp.zeros((pad,), jnp.int32)])
    g = _gram(W1, W2)
    total = None
    for q in range(CHUNKS):
        xl, xr = _make_sc_gather(q * CM)(x, left, right)
        p = _tc_partial(xl, xr, g, q * CM, masked=(q == CHUNKS - 1))
        total = p if total is None else total + p
    return (total * (1.0 / M))[0]
